# baseline (speedup 1.0000x reference)
import functools

import jax
import jax.numpy as jnp
from jax.experimental import pallas as pl
from jax.experimental.pallas import tpu as pltpu

N_TASKS = 10
D_MODEL = 4096
HIDDEN = 128
CLASSES = 3
BATCH = 1024

_BLOCK = 128
_NCHUNK = 4
_CHUNK = D_MODEL // _NCHUNK


def _fused_kernel(x_ref, keys_ref, temp_ref, w1_hbm, b1_ref, w2_ref, b2_ref,
                  tl_ref, tp_ref, logits_ref,
                  x_vmem, w1_vmem, keysn_scr, counts_scr, sems):
    i = pl.program_id(0)
    nb = pl.num_programs(0)

    @pl.when(i == 0)
    def _():
        keys = keys_ref[...]
        ssk = jnp.sum(keys * keys, axis=-1, keepdims=True)
        keysn_scr[...] = keys * jax.lax.rsqrt(jnp.maximum(ssk, 1e-12))
        counts_scr[...] = jnp.zeros_like(counts_scr)

    xb = x_ref[...]
    x_vmem[pl.ds(i * _BLOCK, _BLOCK), :] = xb.astype(jnp.bfloat16)

    ssx = jnp.sum(xb * xb, axis=-1, keepdims=True)
    xinv = jax.lax.rsqrt(jnp.maximum(ssx, 1e-12))
    sim = jax.lax.dot_general(
        xb, keysn_scr[...], (((1,), (1,)), ((), ())),
        preferred_element_type=jnp.float32) * xinv
    logits = sim / temp_ref[0]
    tl_ref[...] = logits

    m = jnp.max(logits, axis=-1, keepdims=True)
    e = jnp.exp(logits - m)
    tp_ref[...] = e / jnp.sum(e, axis=-1, keepdims=True)

    col = jax.lax.broadcasted_iota(jnp.int32, logits.shape, 1)
    pred = jnp.min(jnp.where(logits == m, col, N_TASKS), axis=-1,
                   keepdims=True)
    onehot = (pred == jax.lax.broadcasted_iota(
        jnp.int32, (_BLOCK, N_TASKS), 1)).astype(jnp.int32)
    counts_scr[...] += jnp.sum(onehot, axis=0, keepdims=True)

    @pl.when(i == nb - 1)
    def _():
        counts = counts_scr[...]
        mc = jnp.max(counts)
        tcol = jax.lax.broadcasted_iota(jnp.int32, counts.shape, 1)
        t = jnp.min(jnp.where(counts == mc, tcol, N_TASKS))

        for k in range(_NCHUNK):
            pltpu.make_async_copy(
                w1_hbm.at[t, pl.ds(k * _CHUNK, _CHUNK), :],
                w1_vmem.at[pl.ds(k * _CHUNK, _CHUNK), :],
                sems.at[k]).start()

        trow = jax.lax.broadcasted_iota(jnp.int32, (N_TASKS, 1), 0)
        b1v = jnp.sum(jnp.where(trow == t, b1_ref[...], 0.0), axis=0,
                      keepdims=True)
        b2v = jnp.sum(jnp.where(trow == t, b2_ref[...], 0.0), axis=0,
                      keepdims=True)
        trow3 = jax.lax.broadcasted_iota(jnp.int32, (N_TASKS, 1, 1), 0)
        w2 = jnp.sum(jnp.where(trow3 == t, w2_ref[...], 0.0), axis=0)

        h = jnp.broadcast_to(b1v, (BATCH, HIDDEN))
        for k in range(_NCHUNK):
            pltpu.make_async_copy(
                w1_hbm.at[t, pl.ds(k * _CHUNK, _CHUNK), :],
                w1_vmem.at[pl.ds(k * _CHUNK, _CHUNK), :],
                sems.at[k]).wait()
            w1k = w1_vmem[pl.ds(k * _CHUNK, _CHUNK), :].astype(jnp.bfloat16)
            xk = x_vmem[:, pl.ds(k * _CHUNK, _CHUNK)]
            h = h + jnp.dot(xk, w1k, preferred_element_type=jnp.float32)
        h = jnp.maximum(h, 0.0)
        logits_ref[...] = jnp.dot(
            h.astype(jnp.bfloat16), w2.astype(jnp.bfloat16),
            preferred_element_type=jnp.float32) + b2v


@functools.partial(jax.jit)
def kernel(x, task_keys, temperature, W1, b1, W2, b2):
    nb = BATCH // _BLOCK
    task_logits, task_probs, logits = pl.pallas_call(
        _fused_kernel,
        grid=(nb,),
        in_specs=[
            pl.BlockSpec((_BLOCK, D_MODEL), lambda i: (i, 0)),
            pl.BlockSpec((N_TASKS, D_MODEL), lambda i: (0, 0)),
            pl.BlockSpec(memory_space=pltpu.SMEM),
            pl.BlockSpec(memory_space=pltpu.MemorySpace.HBM),
            pl.BlockSpec((N_TASKS, HIDDEN), lambda i: (0, 0)),
            pl.BlockSpec((N_TASKS, HIDDEN, CLASSES), lambda i: (0, 0, 0)),
            pl.BlockSpec((N_TASKS, CLASSES), lambda i: (0, 0)),
        ],
        out_specs=[
            pl.BlockSpec((_BLOCK, N_TASKS), lambda i: (i, 0)),
            pl.BlockSpec((_BLOCK, N_TASKS), lambda i: (i, 0)),
            pl.BlockSpec((BATCH, CLASSES), lambda i: (0, 0)),
        ],
        out_shape=[
            jax.ShapeDtypeStruct((BATCH, N_TASKS), jnp.float32),
            jax.ShapeDtypeStruct((BATCH, N_TASKS), jnp.float32),
            jax.ShapeDtypeStruct((BATCH, CLASSES), jnp.float32),
        ],
        scratch_shapes=[
            pltpu.VMEM((BATCH, D_MODEL), jnp.bfloat16),
            pltpu.VMEM((D_MODEL, HIDDEN), jnp.float32),
            pltpu.VMEM((N_TASKS, D_MODEL), jnp.float32),
            pltpu.VMEM((1, N_TASKS), jnp.int32),
            pltpu.SemaphoreType.DMA((_NCHUNK,)),
        ],
    )(x, task_keys, temperature, W1, b1, W2, b2)

    return (logits, task_logits, task_probs)


# speedup vs baseline: 1.1024x; 1.1024x over previous
import functools

import jax
import jax.numpy as jnp
from jax.experimental import pallas as pl
from jax.experimental.pallas import tpu as pltpu

N_TASKS = 10
D_MODEL = 4096
HIDDEN = 128
CLASSES = 3
BATCH = 1024

_BLOCK = 256
_NCHUNK = 4
_CHUNK = D_MODEL // _NCHUNK


def _fused_kernel(x_ref, keys_ref, temp_ref, w1_hbm, b1_ref, w2_ref, b2_ref,
                  tl_ref, tp_ref, logits_ref,
                  x_vmem, w1_vmem, keysn_scr, counts_scr, sems):
    i = pl.program_id(0)
    nb = pl.num_programs(0)

    @pl.when(i == 0)
    def _():
        keys = keys_ref[...]
        ssk = jnp.sum(keys * keys, axis=-1, keepdims=True)
        keysn_scr[...] = keys * jax.lax.rsqrt(jnp.maximum(ssk, 1e-12))
        counts_scr[...] = jnp.zeros_like(counts_scr)

    xb = x_ref[...]
    x_vmem[pl.ds(i * _BLOCK, _BLOCK), :] = xb.astype(jnp.bfloat16)

    ssx = jnp.sum(xb * xb, axis=-1, keepdims=True)
    xinv = jax.lax.rsqrt(jnp.maximum(ssx, 1e-12))
    sim = jax.lax.dot_general(
        xb, keysn_scr[...], (((1,), (1,)), ((), ())),
        preferred_element_type=jnp.float32) * xinv
    logits = sim / temp_ref[0]
    tl_ref[...] = logits

    m = jnp.max(logits, axis=-1, keepdims=True)
    e = jnp.exp(logits - m)
    tp_ref[...] = e / jnp.sum(e, axis=-1, keepdims=True)

    col = jax.lax.broadcasted_iota(jnp.int32, logits.shape, 1)
    pred = jnp.min(jnp.where(logits == m, col, N_TASKS), axis=-1,
                   keepdims=True)
    onehot = (pred == jax.lax.broadcasted_iota(
        jnp.int32, (_BLOCK, N_TASKS), 1)).astype(jnp.int32)
    counts_scr[...] += jnp.sum(onehot, axis=0, keepdims=True)

    @pl.when(i == nb - 1)
    def _():
        counts = counts_scr[...]
        mc = jnp.max(counts)
        tcol = jax.lax.broadcasted_iota(jnp.int32, counts.shape, 1)
        t = jnp.min(jnp.where(counts == mc, tcol, N_TASKS))

        for k in range(_NCHUNK):
            pltpu.make_async_copy(
                w1_hbm.at[t, pl.ds(k * _CHUNK, _CHUNK), :],
                w1_vmem.at[pl.ds(k * _CHUNK, _CHUNK), :],
                sems.at[k]).start()

        trow = jax.lax.broadcasted_iota(jnp.int32, (N_TASKS, 1), 0)
        b1v = jnp.sum(jnp.where(trow == t, b1_ref[...], 0.0), axis=0,
                      keepdims=True)
        b2v = jnp.sum(jnp.where(trow == t, b2_ref[...], 0.0), axis=0,
                      keepdims=True)
        trow3 = jax.lax.broadcasted_iota(jnp.int32, (N_TASKS, 1, 1), 0)
        w2 = jnp.sum(jnp.where(trow3 == t, w2_ref[...], 0.0), axis=0)

        h = jnp.broadcast_to(b1v, (BATCH, HIDDEN))
        for k in range(_NCHUNK):
            pltpu.make_async_copy(
                w1_hbm.at[t, pl.ds(k * _CHUNK, _CHUNK), :],
                w1_vmem.at[pl.ds(k * _CHUNK, _CHUNK), :],
                sems.at[k]).wait()
            w1k = w1_vmem[pl.ds(k * _CHUNK, _CHUNK), :].astype(jnp.bfloat16)
            xk = x_vmem[:, pl.ds(k * _CHUNK, _CHUNK)]
            h = h + jnp.dot(xk, w1k, preferred_element_type=jnp.float32)
        h = jnp.maximum(h, 0.0)
        logits_ref[...] = jnp.dot(
            h.astype(jnp.bfloat16), w2.astype(jnp.bfloat16),
            preferred_element_type=jnp.float32) + b2v


@functools.partial(jax.jit)
def kernel(x, task_keys, temperature, W1, b1, W2, b2):
    nb = BATCH // _BLOCK
    task_logits, task_probs, logits = pl.pallas_call(
        _fused_kernel,
        grid=(nb,),
        in_specs=[
            pl.BlockSpec((_BLOCK, D_MODEL), lambda i: (i, 0)),
            pl.BlockSpec((N_TASKS, D_MODEL), lambda i: (0, 0)),
            pl.BlockSpec(memory_space=pltpu.SMEM),
            pl.BlockSpec(memory_space=pltpu.MemorySpace.HBM),
            pl.BlockSpec((N_TASKS, HIDDEN), lambda i: (0, 0)),
            pl.BlockSpec((N_TASKS, HIDDEN, CLASSES), lambda i: (0, 0, 0)),
            pl.BlockSpec((N_TASKS, CLASSES), lambda i: (0, 0)),
        ],
        out_specs=[
            pl.BlockSpec((_BLOCK, N_TASKS), lambda i: (i, 0)),
            pl.BlockSpec((_BLOCK, N_TASKS), lambda i: (i, 0)),
            pl.BlockSpec((BATCH, CLASSES), lambda i: (0, 0)),
        ],
        out_shape=[
            jax.ShapeDtypeStruct((BATCH, N_TASKS), jnp.float32),
            jax.ShapeDtypeStruct((BATCH, N_TASKS), jnp.float32),
            jax.ShapeDtypeStruct((BATCH, CLASSES), jnp.float32),
        ],
        scratch_shapes=[
            pltpu.VMEM((BATCH, D_MODEL), jnp.bfloat16),
            pltpu.VMEM((D_MODEL, HIDDEN), jnp.float32),
            pltpu.VMEM((N_TASKS, D_MODEL), jnp.float32),
            pltpu.VMEM((1, N_TASKS), jnp.int32),
            pltpu.SemaphoreType.DMA((_NCHUNK,)),
        ],
    )(x, task_keys, temperature, W1, b1, W2, b2)

    return (logits, task_logits, task_probs)


# speedup vs baseline: 1.2509x; 1.1347x over previous
import functools

import jax
import jax.numpy as jnp
from jax.experimental import pallas as pl
from jax.experimental.pallas import tpu as pltpu

N_TASKS = 10
D_MODEL = 4096
HIDDEN = 128
CLASSES = 3
BATCH = 1024

_BLOCK = 256
_NB = BATCH // _BLOCK


def _fused_kernel(x_hbm, keys_ref, temp_ref, w1_hbm, b1_ref, w2_ref, b2_ref,
                  tl_ref, tp_ref, logits_ref,
                  x_vmem, w1_vmem, keysn_scr, counts_scr, lead_scr,
                  xsems, wsems):
    i = pl.program_id(0)

    @pl.when(i == 0)
    def _():
        for k in range(_NB):
            pltpu.make_async_copy(
                x_hbm.at[pl.ds(k * _BLOCK, _BLOCK), :],
                x_vmem.at[pl.ds(k * _BLOCK, _BLOCK), :],
                xsems.at[k]).start()
        keys = keys_ref[...]
        ssk = jnp.sum(keys * keys, axis=-1, keepdims=True)
        keysn_scr[...] = keys * jax.lax.rsqrt(jnp.maximum(ssk, 1e-12))
        counts_scr[...] = jnp.zeros_like(counts_scr)

    pltpu.make_async_copy(
        x_hbm.at[pl.ds(i * _BLOCK, _BLOCK), :],
        x_vmem.at[pl.ds(i * _BLOCK, _BLOCK), :],
        xsems.at[i]).wait()
    xb = x_vmem[pl.ds(i * _BLOCK, _BLOCK), :]

    ssx = jnp.sum(xb * xb, axis=-1, keepdims=True)
    xinv = jax.lax.rsqrt(jnp.maximum(ssx, 1e-12))
    sim = jax.lax.dot_general(
        xb, keysn_scr[...], (((1,), (1,)), ((), ())),
        preferred_element_type=jnp.float32) * xinv
    logits = sim / temp_ref[0]
    tl_ref[...] = logits

    m = jnp.max(logits, axis=-1, keepdims=True)
    e = jnp.exp(logits - m)
    tp_ref[...] = e / jnp.sum(e, axis=-1, keepdims=True)

    col = jax.lax.broadcasted_iota(jnp.int32, logits.shape, 1)
    pred = jnp.min(jnp.where(logits == m, col, N_TASKS), axis=-1,
                   keepdims=True)
    onehot = (pred == jax.lax.broadcasted_iota(
        jnp.int32, (_BLOCK, N_TASKS), 1)).astype(jnp.int32)
    counts_scr[...] += jnp.sum(onehot, axis=0, keepdims=True)

    tcol = jax.lax.broadcasted_iota(jnp.int32, (1, N_TASKS), 1)

    @pl.when(i == _NB - 2)
    def _():
        counts = counts_scr[...]
        mc = jnp.max(counts)
        lead = jnp.min(jnp.where(counts == mc, tcol, N_TASKS))
        lead_scr[0] = lead
        pltpu.make_async_copy(w1_hbm.at[lead], w1_vmem, wsems.at[0]).start()

    @pl.when(i == _NB - 1)
    def _():
        counts = counts_scr[...]
        mc = jnp.max(counts)
        t = jnp.min(jnp.where(counts == mc, tcol, N_TASKS))

        lead = lead_scr[0]
        pltpu.make_async_copy(w1_hbm.at[lead], w1_vmem, wsems.at[0]).wait()

        @pl.when(t != lead)
        def _():
            cp = pltpu.make_async_copy(w1_hbm.at[t], w1_vmem, wsems.at[1])
            cp.start()
            cp.wait()

        trow = jax.lax.broadcasted_iota(jnp.int32, (N_TASKS, 1), 0)
        b1v = jnp.sum(jnp.where(trow == t, b1_ref[...], 0.0), axis=0,
                      keepdims=True)
        b2v = jnp.sum(jnp.where(trow == t, b2_ref[...], 0.0), axis=0,
                      keepdims=True)
        trow3 = jax.lax.broadcasted_iota(jnp.int32, (N_TASKS, 1, 1), 0)
        w2 = jnp.sum(jnp.where(trow3 == t, w2_ref[...], 0.0), axis=0)

        h = jnp.dot(x_vmem[...], w1_vmem[...],
                    preferred_element_type=jnp.float32) + b1v
        h = jnp.maximum(h, 0.0)
        logits_ref[...] = jnp.dot(
            h, w2, preferred_element_type=jnp.float32) + b2v


@functools.partial(jax.jit)
def kernel(x, task_keys, temperature, W1, b1, W2, b2):
    task_logits, task_probs, logits = pl.pallas_call(
        _fused_kernel,
        grid=(_NB,),
        in_specs=[
            pl.BlockSpec(memory_space=pltpu.MemorySpace.HBM),
            pl.BlockSpec((N_TASKS, D_MODEL), lambda i: (0, 0)),
            pl.BlockSpec(memory_space=pltpu.SMEM),
            pl.BlockSpec(memory_space=pltpu.MemorySpace.HBM),
            pl.BlockSpec((N_TASKS, HIDDEN), lambda i: (0, 0)),
            pl.BlockSpec((N_TASKS, HIDDEN, CLASSES), lambda i: (0, 0, 0)),
            pl.BlockSpec((N_TASKS, CLASSES), lambda i: (0, 0)),
        ],
        out_specs=[
            pl.BlockSpec((_BLOCK, N_TASKS), lambda i: (i, 0)),
            pl.BlockSpec((_BLOCK, N_TASKS), lambda i: (i, 0)),
            pl.BlockSpec((BATCH, CLASSES), lambda i: (0, 0)),
        ],
        out_shape=[
            jax.ShapeDtypeStruct((BATCH, N_TASKS), jnp.float32),
            jax.ShapeDtypeStruct((BATCH, N_TASKS), jnp.float32),
            jax.ShapeDtypeStruct((BATCH, CLASSES), jnp.float32),
        ],
        scratch_shapes=[
            pltpu.VMEM((BATCH, D_MODEL), jnp.float32),
            pltpu.VMEM((D_MODEL, HIDDEN), jnp.float32),
            pltpu.VMEM((N_TASKS, D_MODEL), jnp.float32),
            pltpu.VMEM((1, N_TASKS), jnp.int32),
            pltpu.SMEM((1,), jnp.int32),
            pltpu.SemaphoreType.DMA((_NB,)),
            pltpu.SemaphoreType.DMA((2,)),
        ],
    )(x, task_keys, temperature, W1, b1, W2, b2)

    return (logits, task_logits, task_probs)
